# baseline (speedup 1.0000x reference)
import functools

import jax
import jax.numpy as jnp
from jax import lax
from jax.experimental import pallas as pl
from jax.experimental.pallas import tpu as pltpu
from jax.experimental.pallas import tpu_sc as plsc

EMB = 16
B = 16384
CTX = 20

NC = 2
NS = 16
NW = NC * NS
BPW = B // NW
CB = 128
NCHUNK = BPW // CB
GROUPS = CB // 16


def _sc_scores(pos_u, pos_w, neg_u, neg_w, U, W):
    mesh = plsc.VectorSubcoreMesh(core_axis_name="c", subcore_axis_name="s")

    buf = lambda shape, dt: [pltpu.VMEM(shape, dt), pltpu.VMEM(shape, dt)]

    @functools.partial(
        pl.kernel,
        out_type=(
            jax.ShapeDtypeStruct((B,), jnp.float32),
            jax.ShapeDtypeStruct((B,), jnp.float32),
        ),
        mesh=mesh,
        compiler_params=pltpu.CompilerParams(
            needs_layout_passes=False, use_tc_tiling_on_sc=False),
        scratch_types=[
            buf((CB * CTX,), jnp.int32),
            buf((CB * CTX, EMB), jnp.float32),
            buf((CB,), jnp.int32),
            buf((CB, EMB), jnp.float32),
            pltpu.VMEM((CB * EMB,), jnp.float32),
            pltpu.VMEM((CB,), jnp.float32),
            [pltpu.SemaphoreType.DMA, pltpu.SemaphoreType.DMA],
            [pltpu.SemaphoreType.DMA, pltpu.SemaphoreType.DMA],
        ],
    )
    def k(pos_u_h, pos_w_h, neg_u_h, neg_w_h, U_h, W_h, out_p, out_n,
          uidx2, rows2, widx2, wrows2, prod, scores, semu2, semw2):
        wid = lax.axis_index("s") * NC + lax.axis_index("c")
        base = wid * BPW
        ROUNDS = 2 * NCHUNK

        def round_refs(r):
            side, ci = r // NCHUNK, r % NCHUNK
            uh = (pos_u_h, neg_u_h)[side]
            wh = (pos_w_h, neg_w_h)[side]
            oh = (out_p, out_n)[side]
            return uh, wh, oh, base + ci * CB

        def stage(r, p):
            uh, wh, _, gb = round_refs(r)
            pltpu.sync_copy(
                uh.at[pl.ds(pl.multiple_of(gb * CTX, 8), CB * CTX)], uidx2[p])
            pltpu.sync_copy(wh.at[pl.ds(pl.multiple_of(gb, CB), CB)], widx2[p])
            pltpu.async_copy(U_h.at[uidx2[p]], rows2[p], semu2[p])
            pltpu.async_copy(W_h.at[widx2[p]], wrows2[p], semw2[p])

        def compute(r, p):
            _, _, oh, gb = round_refs(r)
            rows, wrows = rows2[p], wrows2[p]
            pltpu.make_async_copy(U_h.at[uidx2[p]], rows, semu2[p]).wait()
            pltpu.make_async_copy(W_h.at[widx2[p]], wrows, semw2[p]).wait()

            def bag(b, _):
                r0 = b * CTX
                acc = rows[r0, :]
                for e in range(1, CTX):
                    acc = acc + rows[r0 + e, :]
                prod[pl.ds(pl.multiple_of(b * EMB, 8), EMB)] = acc * wrows[b, :]
                return 0

            lax.fori_loop(0, CB, bag, 0)

            def dot(g, _):
                bvec = jnp.int32(16) * g + lax.iota(jnp.int32, 16)
                fvec = bvec * EMB
                sacc = jnp.zeros((16,), jnp.float32)
                for e in range(EMB):
                    sacc = sacc + plsc.load_gather(prod, [fvec + e])
                scores[pl.ds(g * 16, 16)] = sacc
                return 0

            lax.fori_loop(0, GROUPS, dot, 0)
            pltpu.sync_copy(scores, oh.at[pl.ds(pl.multiple_of(gb, CB), CB)])

        stage(0, 0)
        for r in range(ROUNDS):
            if r + 1 < ROUNDS:
                stage(r + 1, (r + 1) % 2)
            compute(r, r % 2)

    return k(pos_u, pos_w, neg_u, neg_w, U, W)


TCOL = 98304
DT_GRID = (1999999 + TCOL - 1) // TCOL
TABLE_PAD = DT_GRID * TCOL
OROWS = TABLE_PAD * EMB // 128


def _detranspose_tc(Ut, Wt):
    def body(u_ref, w_ref, uo_ref, wo_ref):
        for ref, oref in ((u_ref, uo_ref), (w_ref, wo_ref)):
            for q in range(TCOL // 1024):
                stacked = jnp.concatenate(
                    [ref[:, pl.ds(1024 * q + 128 * j, 128)] for j in range(8)],
                    axis=0)
                oref[pl.ds(128 * q, 128), :] = stacked.T

    uo, wo = pl.pallas_call(
        body,
        grid=(DT_GRID,),
        in_specs=[
            pl.BlockSpec((EMB, TCOL), lambda i: (0, i)),
            pl.BlockSpec((EMB, TCOL), lambda i: (0, i)),
        ],
        out_specs=[
            pl.BlockSpec((TCOL * EMB // 128, 128), lambda i: (i, 0)),
            pl.BlockSpec((TCOL * EMB // 128, 128), lambda i: (i, 0)),
        ],
        out_shape=[
            jax.ShapeDtypeStruct((OROWS, 128), jnp.float32),
            jax.ShapeDtypeStruct((OROWS, 128), jnp.float32),
        ],
    )(Ut, Wt)
    return (uo.reshape(TABLE_PAD, EMB), wo.reshape(TABLE_PAD, EMB))


def _remap_idx(i):
    return (i & ~1023) + ((i & 127) << 3) + ((i >> 7) & 7)


def _loss_tc(sp, sn):
    def body(sp_ref, sn_ref, out_ref):
        pos = sp_ref[...]
        neg = sn_ref[...]
        lp = jnp.minimum(pos, 0.0) - jnp.log1p(jnp.exp(-jnp.abs(pos)))
        ln = jnp.minimum(-neg, 0.0) - jnp.log1p(jnp.exp(-jnp.abs(neg)))
        out_ref[0, 0] = -(jnp.sum(lp) + jnp.sum(ln))

    out = pl.pallas_call(
        body,
        out_shape=jax.ShapeDtypeStruct((1, 1), jnp.float32),
        out_specs=pl.BlockSpec(memory_space=pltpu.SMEM),
    )(sp.reshape(128, 128), sn.reshape(128, 128))
    return out[0, 0]


def kernel(pos_u, pos_w, neg_u, neg_w, U, W):
    pu = _remap_idx(jnp.asarray(pos_u, jnp.int32).reshape(B * CTX))
    nu = _remap_idx(jnp.asarray(neg_u, jnp.int32).reshape(B * CTX))
    pw = _remap_idx(jnp.asarray(pos_w, jnp.int32))
    nw = _remap_idx(jnp.asarray(neg_w, jnp.int32))
    U_lin, W_lin = _detranspose_tc(U.T, W.T)
    sp, sn = _sc_scores(pu, pw, nu, nw, U_lin, W_lin)
    return _loss_tc(sp, sn)


# speedup vs baseline: 1.0011x; 1.0011x over previous
import functools

import jax
import jax.numpy as jnp
from jax import lax
from jax.experimental import pallas as pl
from jax.experimental.pallas import tpu as pltpu
from jax.experimental.pallas import tpu_sc as plsc

EMB = 16
B = 16384
CTX = 20

NC = 2
NS = 16
NW = NC * NS
BPW = B // NW
CB = 128
NCHUNK = BPW // CB
GROUPS = CB // 16


def _sc_scores(pos_u, pos_w, neg_u, neg_w, U, W):
    mesh = plsc.VectorSubcoreMesh(core_axis_name="c", subcore_axis_name="s")

    buf = lambda shape, dt: [pltpu.VMEM(shape, dt), pltpu.VMEM(shape, dt)]

    @functools.partial(
        pl.kernel,
        out_type=(
            jax.ShapeDtypeStruct((B,), jnp.float32),
            jax.ShapeDtypeStruct((B,), jnp.float32),
        ),
        mesh=mesh,
        compiler_params=pltpu.CompilerParams(
            needs_layout_passes=False, use_tc_tiling_on_sc=False),
        scratch_types=[
            buf((CB * CTX,), jnp.int32),
            buf((CB * CTX, EMB), jnp.float32),
            buf((CB,), jnp.int32),
            buf((CB, EMB), jnp.float32),
            pltpu.VMEM((CB * EMB,), jnp.float32),
            pltpu.VMEM((CB,), jnp.float32),
            [pltpu.SemaphoreType.DMA, pltpu.SemaphoreType.DMA],
            [pltpu.SemaphoreType.DMA, pltpu.SemaphoreType.DMA],
        ],
    )
    def k(pos_u_h, pos_w_h, neg_u_h, neg_w_h, U_h, W_h, out_p, out_n,
          uidx2, rows2, widx2, wrows2, prod, scores, semu2, semw2):
        wid = lax.axis_index("s") * NC + lax.axis_index("c")
        base = wid * BPW
        ROUNDS = 2 * NCHUNK

        def round_refs(r):
            side, ci = r // NCHUNK, r % NCHUNK
            uh = (pos_u_h, neg_u_h)[side]
            wh = (pos_w_h, neg_w_h)[side]
            oh = (out_p, out_n)[side]
            return uh, wh, oh, base + ci * CB

        def stage(r, p):
            uh, wh, _, gb = round_refs(r)
            pltpu.sync_copy(
                uh.at[pl.ds(pl.multiple_of(gb * CTX, 8), CB * CTX)], uidx2[p])
            pltpu.sync_copy(wh.at[pl.ds(pl.multiple_of(gb, CB), CB)], widx2[p])
            pltpu.async_copy(U_h.at[uidx2[p]], rows2[p], semu2[p])
            pltpu.async_copy(W_h.at[widx2[p]], wrows2[p], semw2[p])

        def compute(r, p):
            _, _, oh, gb = round_refs(r)
            rows, wrows = rows2[p], wrows2[p]
            pltpu.make_async_copy(U_h.at[uidx2[p]], rows, semu2[p]).wait()
            pltpu.make_async_copy(W_h.at[widx2[p]], wrows, semw2[p]).wait()

            def bag(b, _):
                r0 = b * CTX
                acc = rows[r0, :]
                for e in range(1, CTX):
                    acc = acc + rows[r0 + e, :]
                prod[pl.ds(pl.multiple_of(b * EMB, 8), EMB)] = acc * wrows[b, :]
                return 0

            lax.fori_loop(0, CB, bag, 0)

            def dot(g, _):
                bvec = jnp.int32(16) * g + lax.iota(jnp.int32, 16)
                fvec = bvec * EMB
                sacc = jnp.zeros((16,), jnp.float32)
                for e in range(EMB):
                    sacc = sacc + plsc.load_gather(prod, [fvec + e])
                scores[pl.ds(g * 16, 16)] = sacc
                return 0

            lax.fori_loop(0, GROUPS, dot, 0)
            pltpu.sync_copy(scores, oh.at[pl.ds(pl.multiple_of(gb, CB), CB)])

        stage(0, 0)
        for r in range(ROUNDS):
            if r + 1 < ROUNDS:
                stage(r + 1, (r + 1) % 2)
            compute(r, r % 2)

    return k(pos_u, pos_w, neg_u, neg_w, U, W)


TCOL = 65536
DT_GRID = (1999999 + TCOL - 1) // TCOL
TABLE_PAD = DT_GRID * TCOL
OROWS = TABLE_PAD * EMB // 128


def _detranspose_tc(Ut, Wt):
    def body(u_ref, w_ref, uo_ref, wo_ref):
        for ref, oref in ((u_ref, uo_ref), (w_ref, wo_ref)):
            for q in range(TCOL // 1024):
                stacked = jnp.concatenate(
                    [ref[:, pl.ds(1024 * q + 128 * j, 128)] for j in range(8)],
                    axis=0)
                oref[pl.ds(128 * q, 128), :] = stacked.T

    uo, wo = pl.pallas_call(
        body,
        grid=(DT_GRID,),
        in_specs=[
            pl.BlockSpec((EMB, TCOL), lambda i: (0, i)),
            pl.BlockSpec((EMB, TCOL), lambda i: (0, i)),
        ],
        out_specs=[
            pl.BlockSpec((TCOL * EMB // 128, 128), lambda i: (i, 0)),
            pl.BlockSpec((TCOL * EMB // 128, 128), lambda i: (i, 0)),
        ],
        out_shape=[
            jax.ShapeDtypeStruct((OROWS, 128), jnp.float32),
            jax.ShapeDtypeStruct((OROWS, 128), jnp.float32),
        ],
    )(Ut, Wt)
    return (uo.reshape(TABLE_PAD, EMB), wo.reshape(TABLE_PAD, EMB))


def _remap_idx(i):
    return (i & ~1023) + ((i & 127) << 3) + ((i >> 7) & 7)


def _loss_tc(sp, sn):
    def body(sp_ref, sn_ref, out_ref):
        pos = sp_ref[...]
        neg = sn_ref[...]
        lp = jnp.minimum(pos, 0.0) - jnp.log1p(jnp.exp(-jnp.abs(pos)))
        ln = jnp.minimum(-neg, 0.0) - jnp.log1p(jnp.exp(-jnp.abs(neg)))
        out_ref[0, 0] = -(jnp.sum(lp) + jnp.sum(ln))

    out = pl.pallas_call(
        body,
        out_shape=jax.ShapeDtypeStruct((1, 1), jnp.float32),
        out_specs=pl.BlockSpec(memory_space=pltpu.SMEM),
    )(sp.reshape(128, 128), sn.reshape(128, 128))
    return out[0, 0]


def kernel(pos_u, pos_w, neg_u, neg_w, U, W):
    pu = _remap_idx(jnp.asarray(pos_u, jnp.int32).reshape(B * CTX))
    nu = _remap_idx(jnp.asarray(neg_u, jnp.int32).reshape(B * CTX))
    pw = _remap_idx(jnp.asarray(pos_w, jnp.int32))
    nw = _remap_idx(jnp.asarray(neg_w, jnp.int32))
    U_lin, W_lin = _detranspose_tc(U.T, W.T)
    sp, sn = _sc_scores(pu, pw, nu, nw, U_lin, W_lin)
    return _loss_tc(sp, sn)
